# baseline (speedup 1.0000x reference)
import functools

import jax
import jax.numpy as jnp
from jax import lax
from jax.experimental import pallas as pl
from jax.experimental.pallas import tpu as pltpu
from jax.experimental.pallas import tpu_sc as plsc

NUM_WORKERS = 32
B_TOTAL = 4096 * 50
D = 64
B_PER_W = B_TOTAL // NUM_WORKERS
CHUNK = 128
N_CHUNKS = B_PER_W // CHUNK


def _emb_body(table, idx, out, idx_v, f0, f1, g0, g1, o0, o1):
    w = lax.axis_index("s") * 2 + lax.axis_index("c")
    base = w * B_PER_W

    pltpu.sync_copy(idx.at[w], idx_v)

    fbuf = [f0, f1]
    gsem = [g0, g1]
    osem = [o0, o1]

    def start_gather(j, b):
        pltpu.async_copy(table.at[idx_v.at[j]], fbuf[b], gsem[b])

    def wait_gather(j, b):
        pltpu.make_async_copy(table.at[idx_v.at[j]], fbuf[b], gsem[b]).wait()

    def start_out(j, b):
        pltpu.async_copy(fbuf[b], out.at[pl.ds(base + j * CHUNK, CHUNK)],
                         osem[b])

    def wait_out(j, b):
        pltpu.make_async_copy(fbuf[b], out.at[pl.ds(base + j * CHUNK, CHUNK)],
                              osem[b]).wait()

    start_gather(0, 0)
    start_gather(1, 1)
    wait_gather(0, 0)
    start_out(0, 0)

    def pair_body(i, _):
        for parity in range(2):
            j = 2 * i + 1 + parity
            b = (1 + parity) % 2
            wait_out(j - 1, 1 - b)
            start_gather(j + 1, 1 - b)
            wait_gather(j, b)
            start_out(j, b)
        return 0

    lax.fori_loop(0, 24, pair_body, 0)

    wait_out(48, 0)
    wait_gather(49, 1)
    start_out(49, 1)
    wait_out(49, 1)


_emb = functools.partial(
    pl.kernel,
    out_type=jax.ShapeDtypeStruct((B_TOTAL, D), jnp.bfloat16),
    mesh=plsc.VectorSubcoreMesh(core_axis_name="c", subcore_axis_name="s"),
    scratch_types=[
        pltpu.VMEM((N_CHUNKS, CHUNK), jnp.int32),
        pltpu.VMEM((CHUNK, D), jnp.bfloat16),
        pltpu.VMEM((CHUNK, D), jnp.bfloat16),
        pltpu.SemaphoreType.DMA,
        pltpu.SemaphoreType.DMA,
        pltpu.SemaphoreType.DMA,
        pltpu.SemaphoreType.DMA,
    ],
    compiler_params=pltpu.CompilerParams(
        needs_layout_passes=False,
        use_tc_tiling_on_sc=False,
    ),
)(_emb_body)


def kernel(input, weight):
    table = weight.astype(jnp.bfloat16) * jnp.asarray(8.0, dtype=jnp.bfloat16)
    idx = jnp.reshape(input, (NUM_WORKERS, N_CHUNKS, CHUNK))
    out = _emb(table, idx)
    return out.reshape(input.shape[0], input.shape[1], D)
